# baseline (speedup 1.0000x reference)
import functools

import jax
import jax.numpy as jnp
from jax import lax
from jax.experimental import pallas as pl
from jax.experimental.pallas import tpu as pltpu
from jax.experimental.pallas import tpu_sc as plsc

NC = 2
NS = 16
NW = NC * NS
CHUNK = 64
RING = 3


def _pre_body(x_ref, ws_ref, wt_ref, bs_ref, bv_ref, s_ref, y_ref):
  xb = x_ref[...]
  s_ref[...] = jnp.dot(xb, ws_ref[...], preferred_element_type=jnp.float32) + bs_ref[...]
  y_ref[...] = jnp.dot(xb, wt_ref[...], preferred_element_type=jnp.float32) + bv_ref[...]


def _pre(x, W_S, W_T, b_S, b_vec, block_rows):
  n, d = x.shape
  grid = (n // block_rows,)
  out = jax.ShapeDtypeStruct((n, d), jnp.float32)
  return pl.pallas_call(
      _pre_body,
      grid=grid,
      in_specs=[
          pl.BlockSpec((block_rows, d), lambda i: (i, 0)),
          pl.BlockSpec((d, d), lambda i: (0, 0)),
          pl.BlockSpec((d, d), lambda i: (0, 0)),
          pl.BlockSpec((1, d), lambda i: (0, 0)),
          pl.BlockSpec((1, d), lambda i: (0, 0)),
      ],
      out_specs=[
          pl.BlockSpec((block_rows, d), lambda i: (i, 0)),
          pl.BlockSpec((block_rows, d), lambda i: (i, 0)),
      ],
      out_shape=[out, out],
      compiler_params=pltpu.CompilerParams(
          dimension_semantics=("parallel",)),
  )(x, W_S, W_T, b_S, b_vec)


def _sc_agg_body(n_pad, q, y_hbm, row_hbm, col_hbm, tf_hbm,
                 zacc_hbm, zt_hbm, ids_hbm, acc_out, tagg_out,
                 acc_sp, tagg_sp, ids_v,
                 col_v0, col_v1, col_v2, row_v0, row_v1, row_v2,
                 rows_v0, rows_v1, rows_v2, tf_v0, tf_v1, tf_v2,
                 lsem0, lsem1, lsem2, gsem0, gsem1, gsem2,
                 ssem0, ssem1, ssem2):
  c = lax.axis_index("c")
  s = lax.axis_index("s")
  wid = c * NS + s
  col_v = (col_v0, col_v1, col_v2)
  row_v = (row_v0, row_v1, row_v2)
  rows_v = (rows_v0, rows_v1, rows_v2)
  tf_v = (tf_v0, tf_v1, tf_v2)
  lsem = (lsem0, lsem1, lsem2)
  gsem = (gsem0, gsem1, gsem2)
  ssem = (ssem0, ssem1, ssem2)

  nzc = n_pad // CHUNK
  my_nz = (nzc - s + NS - 1) // NS

  @pl.loop(0, my_nz)
  def _zero(k):
    off = (s + k * NS) * CHUNK
    pltpu.sync_copy(ids_hbm.at[pl.ds(off, CHUNK)], ids_v)
    pltpu.sync_copy(zacc_hbm.at[pl.ds(off, CHUNK), :], rows_v0)
    pltpu.sync_copy(zt_hbm.at[pl.ds(off, CHUNK), :], tf_v0)
    pltpu.sync_copy(rows_v0, acc_sp.at[ids_v])
    pltpu.sync_copy(tf_v0, tagg_sp.at[ids_v])

  plsc.subcore_barrier()

  base = wid * q

  def issue_loads(g, b):
    off = (base + g) * CHUNK
    pltpu.async_copy(col_hbm.at[pl.ds(off, CHUNK)], col_v[b], lsem[b])
    pltpu.async_copy(row_hbm.at[pl.ds(off, CHUNK)], row_v[b], lsem[b])
    pltpu.async_copy(tf_hbm.at[pl.ds(off, CHUNK), :], tf_v[b], lsem[b])

  def wait_loads(b):
    pltpu.make_async_copy(col_hbm.at[pl.ds(0, CHUNK)], col_v[b], lsem[b]).wait()
    pltpu.make_async_copy(row_hbm.at[pl.ds(0, CHUNK)], row_v[b], lsem[b]).wait()
    pltpu.make_async_copy(tf_hbm.at[pl.ds(0, CHUNK), :], tf_v[b], lsem[b]).wait()

  def drain_scatters(b):
    pltpu.make_async_copy(rows_v[b], acc_sp.at[row_v[b]], ssem[b]).wait()
    pltpu.make_async_copy(tf_v[b], tagg_sp.at[row_v[b]], ssem[b]).wait()

  issue_loads(0, 0)
  issue_loads(1, 1)

  @pl.loop(0, q // RING)
  def _pipe(p):
    for r in range(RING):
      g = RING * p + r
      b = r
      pb = (r - 1) % RING
      wait_loads(b)
      pltpu.async_copy(y_hbm.at[col_v[b]], rows_v[b], gsem[b])
      if r == 0:
        @pl.when(p > 0)
        def _():
          drain_scatters(pb)
          issue_loads(g + 2, pb)

        @pl.when(p == 0)
        def _():
          issue_loads(g + 2, pb)
      else:
        drain_scatters(pb)

        @pl.when(g + 2 < q)
        def _():
          issue_loads(g + 2, pb)
      pltpu.make_async_copy(y_hbm.at[col_v[b]], rows_v[b], gsem[b]).wait()
      pltpu.async_copy(rows_v[b], acc_sp.at[row_v[b]], ssem[b], add=True)
      pltpu.async_copy(tf_v[b], tagg_sp.at[row_v[b]], ssem[b], add=True)

  drain_scatters((q - 1) % RING)

  plsc.subcore_barrier()

  @pl.loop(0, my_nz)
  def _wout(k):
    off = (s + k * NS) * CHUNK
    pltpu.sync_copy(ids_hbm.at[pl.ds(off, CHUNK)], ids_v)
    pltpu.sync_copy(acc_sp.at[ids_v], rows_v0)
    pltpu.sync_copy(tagg_sp.at[ids_v], tf_v0)
    pltpu.sync_copy(rows_v0, acc_out.at[c, pl.ds(off, CHUNK), :])
    pltpu.sync_copy(tf_v0, tagg_out.at[c, pl.ds(off, CHUNK), :])


def _sc_agg(y, row, col, tf, zacc, zt, ids):
  n_pad, d = zacc.shape
  (e,) = row.shape
  dt = tf.shape[1]
  assert e % (NW * CHUNK * RING) == 0 and n_pad % CHUNK == 0
  q = e // (NW * CHUNK)
  mesh = plsc.VectorSubcoreMesh(core_axis_name="c", subcore_axis_name="s")
  kern = pl.kernel(
      functools.partial(_sc_agg_body, n_pad, q),
      out_type=[
          jax.ShapeDtypeStruct((NC, n_pad, d), jnp.float32),
          jax.ShapeDtypeStruct((NC, n_pad, dt), jnp.float32),
      ],
      mesh=mesh,
      compiler_params=pltpu.CompilerParams(use_tc_tiling_on_sc=False),
      scratch_types=(
          [pltpu.VMEM_SHARED((n_pad, d), jnp.float32),
           pltpu.VMEM_SHARED((n_pad, dt), jnp.float32),
           pltpu.VMEM((CHUNK,), jnp.int32)]
          + [pltpu.VMEM((CHUNK,), jnp.int32)] * 6
          + [pltpu.VMEM((CHUNK, d), jnp.float32)] * 3
          + [pltpu.VMEM((CHUNK, dt), jnp.float32)] * 3
          + [pltpu.SemaphoreType.DMA] * 9
      ),
  )
  return kern(y, row, col, tf, zacc, zt, ids)


def _post_body(s_ref, acc_ref, tagg_ref, wt_ref, o_ref):
  agg = acc_ref[0] + acc_ref[1]
  tg = tagg_ref[0] + tagg_ref[1]
  msg = agg + jnp.dot(tg, wt_ref[...], preferred_element_type=jnp.float32)
  o_ref[...] = jnp.maximum(s_ref[...] + msg, 0.0)


def _post(s, acc, tagg, W_tmp, block_rows):
  n, d = s.shape
  dt = W_tmp.shape[0]
  grid = (n // block_rows,)
  return pl.pallas_call(
      _post_body,
      grid=grid,
      in_specs=[
          pl.BlockSpec((block_rows, d), lambda i: (i, 0)),
          pl.BlockSpec((NC, block_rows, d), lambda i: (0, i, 0)),
          pl.BlockSpec((NC, block_rows, dt), lambda i: (0, i, 0)),
          pl.BlockSpec((dt, d), lambda i: (0, 0)),
      ],
      out_specs=pl.BlockSpec((block_rows, d), lambda i: (i, 0)),
      out_shape=jax.ShapeDtypeStruct((n, d), jnp.float32),
      compiler_params=pltpu.CompilerParams(
          dimension_semantics=("parallel",)),
  )(s, acc, tagg, W_tmp)


def kernel(x, edge_index, temporal_features, W_S, b_S, W_T, b_T, W_tmp, b_tmp):
  n, d = x.shape
  row = edge_index[0].astype(jnp.int32)
  col = edge_index[1].astype(jnp.int32)
  tf = temporal_features.astype(jnp.float32)
  dt = tf.shape[1]
  b_s2 = b_S.reshape(1, d).astype(jnp.float32)
  b_vec = (b_T + b_tmp).reshape(1, d).astype(jnp.float32)

  s_feat, y = _pre(x, W_S, W_T, b_s2, b_vec, block_rows=1000)

  n_pad = ((n + CHUNK) // CHUNK) * CHUNK

  e = row.shape[0]
  stride = NW * CHUNK * RING
  e_pad = ((e + stride - 1) // stride) * stride
  if e_pad != e:
    extra = e_pad - e
    row = jnp.concatenate([row, jnp.full((extra,), n, jnp.int32)])
    col = jnp.concatenate([col, jnp.zeros((extra,), jnp.int32)])
    tf = jnp.concatenate([tf, jnp.zeros((extra, dt), jnp.float32)])

  zacc = jnp.zeros((n_pad, d), jnp.float32)
  zt = jnp.zeros((n_pad, dt), jnp.float32)
  ids = jnp.arange(n_pad, dtype=jnp.int32)
  acc, tagg = _sc_agg(y, row, col, tf, zacc, zt, ids)

  return _post(s_feat, acc, tagg, W_tmp, block_rows=1000)


# speedup vs baseline: 2.2016x; 2.2016x over previous
import functools

import jax
import jax.numpy as jnp
from jax import lax
from jax.experimental import pallas as pl
from jax.experimental.pallas import tpu as pltpu
from jax.experimental.pallas import tpu_sc as plsc

NC = 2
NS = 16
NW = NC * NS
CHUNK = 128
RING = 2


def _pre_body(x_ref, ws_ref, wt_ref, bs_ref, bv_ref, s_ref, y_ref):
  xb = x_ref[...]
  s_ref[...] = jnp.dot(xb, ws_ref[...], preferred_element_type=jnp.float32) + bs_ref[...]
  y_ref[...] = jnp.dot(xb, wt_ref[...], preferred_element_type=jnp.float32) + bv_ref[...]


def _pre(x, W_S, W_T, b_S, b_vec, block_rows):
  n, d = x.shape
  grid = (n // block_rows,)
  out = jax.ShapeDtypeStruct((n, d), jnp.float32)
  return pl.pallas_call(
      _pre_body,
      grid=grid,
      in_specs=[
          pl.BlockSpec((block_rows, d), lambda i: (i, 0)),
          pl.BlockSpec((d, d), lambda i: (0, 0)),
          pl.BlockSpec((d, d), lambda i: (0, 0)),
          pl.BlockSpec((1, d), lambda i: (0, 0)),
          pl.BlockSpec((1, d), lambda i: (0, 0)),
      ],
      out_specs=[
          pl.BlockSpec((block_rows, d), lambda i: (i, 0)),
          pl.BlockSpec((block_rows, d), lambda i: (i, 0)),
      ],
      out_shape=[out, out],
      compiler_params=pltpu.CompilerParams(
          dimension_semantics=("parallel",)),
  )(x, W_S, W_T, b_S, b_vec)


def _sc_agg_body(n_pad, q0, extra, y_hbm, row_hbm, col_hbm, tf_hbm,
                 zacc_hbm, zt_hbm, ids_hbm, acc_out, tagg_out,
                 acc_sp, tagg_sp, ids_v,
                 col_v0, col_v1, row_v0, row_v1,
                 rows_v0, rows_v1, tf_v0, tf_v1,
                 lsem0, lsem1, gsem0, gsem1, ssem0, ssem1):
  c = lax.axis_index("c")
  s = lax.axis_index("s")
  wid = c * NS + s
  col_v = (col_v0, col_v1)
  row_v = (row_v0, row_v1)
  rows_v = (rows_v0, rows_v1)
  tf_v = (tf_v0, tf_v1)
  lsem = (lsem0, lsem1)
  gsem = (gsem0, gsem1)
  ssem = (ssem0, ssem1)

  nzc = n_pad // CHUNK
  my_nz = (nzc - s + NS - 1) // NS

  @pl.loop(0, my_nz)
  def _zero(k):
    off = (s + k * NS) * CHUNK
    pltpu.sync_copy(ids_hbm.at[pl.ds(off, CHUNK)], ids_v)
    pltpu.sync_copy(zacc_hbm.at[pl.ds(off, CHUNK), :], rows_v0)
    pltpu.sync_copy(zt_hbm.at[pl.ds(off, CHUNK), :], tf_v0)
    pltpu.sync_copy(rows_v0, acc_sp.at[ids_v])
    pltpu.sync_copy(tf_v0, tagg_sp.at[ids_v])

  plsc.subcore_barrier()

  base = wid * q0

  def issue_loads(g, b):
    off = (base + g) * CHUNK
    pltpu.async_copy(col_hbm.at[pl.ds(off, CHUNK)], col_v[b], lsem[b])
    pltpu.async_copy(row_hbm.at[pl.ds(off, CHUNK)], row_v[b], lsem[b])
    pltpu.async_copy(tf_hbm.at[pl.ds(off, CHUNK), :], tf_v[b], lsem[b])

  def wait_loads(b):
    pltpu.make_async_copy(col_hbm.at[pl.ds(0, CHUNK)], col_v[b], lsem[b]).wait()
    pltpu.make_async_copy(row_hbm.at[pl.ds(0, CHUNK)], row_v[b], lsem[b]).wait()
    pltpu.make_async_copy(tf_hbm.at[pl.ds(0, CHUNK), :], tf_v[b], lsem[b]).wait()

  def drain_scatters(b):
    pltpu.make_async_copy(rows_v[b], acc_sp.at[row_v[b]], ssem[b]).wait()
    pltpu.make_async_copy(tf_v[b], tagg_sp.at[row_v[b]], ssem[b]).wait()

  issue_loads(0, 0)

  @pl.loop(0, q0 // RING)
  def _pipe(p):
    for r in range(RING):
      g = RING * p + r
      b = r
      ob = 1 - r
      wait_loads(b)
      pltpu.async_copy(y_hbm.at[col_v[b]], rows_v[b], gsem[b])
      if r == 0:
        @pl.when(p > 0)
        def _():
          drain_scatters(ob)
        issue_loads(g + 1, ob)
      else:
        drain_scatters(ob)

        @pl.when(g + 1 < q0)
        def _():
          issue_loads(g + 1, ob)
      pltpu.make_async_copy(y_hbm.at[col_v[b]], rows_v[b], gsem[b]).wait()
      pltpu.async_copy(rows_v[b], acc_sp.at[row_v[b]], ssem[b], add=True)
      pltpu.async_copy(tf_v[b], tagg_sp.at[row_v[b]], ssem[b], add=True)

  drain_scatters((q0 - 1) % RING)

  if extra:
    @pl.when(wid < extra)
    def _extra():
      off = (q0 * NW + wid) * CHUNK
      pltpu.sync_copy(col_hbm.at[pl.ds(off, CHUNK)], col_v0)
      pltpu.sync_copy(row_hbm.at[pl.ds(off, CHUNK)], row_v0)
      pltpu.sync_copy(tf_hbm.at[pl.ds(off, CHUNK), :], tf_v0)
      pltpu.sync_copy(y_hbm.at[col_v0], rows_v0)
      pltpu.sync_copy(rows_v0, acc_sp.at[row_v0], add=True)
      pltpu.sync_copy(tf_v0, tagg_sp.at[row_v0], add=True)

  plsc.subcore_barrier()

  @pl.loop(0, my_nz)
  def _wout(k):
    off = (s + k * NS) * CHUNK
    pltpu.sync_copy(ids_hbm.at[pl.ds(off, CHUNK)], ids_v)
    pltpu.sync_copy(acc_sp.at[ids_v], rows_v0)
    pltpu.sync_copy(tagg_sp.at[ids_v], tf_v0)
    pltpu.sync_copy(rows_v0, acc_out.at[c, pl.ds(off, CHUNK), :])
    pltpu.sync_copy(tf_v0, tagg_out.at[c, pl.ds(off, CHUNK), :])


def _sc_agg(y, row, col, tf, zacc, zt, ids):
  n_pad, d = zacc.shape
  (e,) = row.shape
  dt = tf.shape[1]
  assert e % CHUNK == 0 and n_pad % CHUNK == 0
  nec = e // CHUNK
  q0 = (nec // NW) // RING * RING
  extra = nec - q0 * NW
  assert 0 <= extra < 2 * NW and q0 > 0
  mesh = plsc.VectorSubcoreMesh(core_axis_name="c", subcore_axis_name="s")
  kern = pl.kernel(
      functools.partial(_sc_agg_body, n_pad, q0, extra),
      out_type=[
          jax.ShapeDtypeStruct((NC, n_pad, d), jnp.float32),
          jax.ShapeDtypeStruct((NC, n_pad, dt), jnp.float32),
      ],
      mesh=mesh,
      compiler_params=pltpu.CompilerParams(use_tc_tiling_on_sc=False),
      scratch_types=(
          [pltpu.VMEM_SHARED((n_pad, d), jnp.float32),
           pltpu.VMEM_SHARED((n_pad, dt), jnp.float32),
           pltpu.VMEM((CHUNK,), jnp.int32)]
          + [pltpu.VMEM((CHUNK,), jnp.int32)] * 4
          + [pltpu.VMEM((CHUNK, d), jnp.float32)] * 2
          + [pltpu.VMEM((CHUNK, dt), jnp.float32)] * 2
          + [pltpu.SemaphoreType.DMA] * 6
      ),
  )
  return kern(y, row, col, tf, zacc, zt, ids)


def _post_body(s_ref, acc_ref, tagg_ref, wt_ref, o_ref):
  agg = acc_ref[0] + acc_ref[1]
  tg = tagg_ref[0] + tagg_ref[1]
  msg = agg + jnp.dot(tg, wt_ref[...], preferred_element_type=jnp.float32)
  o_ref[...] = jnp.maximum(s_ref[...] + msg, 0.0)


def _post(s, acc, tagg, W_tmp, block_rows):
  n, d = s.shape
  dt = W_tmp.shape[0]
  grid = (n // block_rows,)
  return pl.pallas_call(
      _post_body,
      grid=grid,
      in_specs=[
          pl.BlockSpec((block_rows, d), lambda i: (i, 0)),
          pl.BlockSpec((NC, block_rows, d), lambda i: (0, i, 0)),
          pl.BlockSpec((NC, block_rows, dt), lambda i: (0, i, 0)),
          pl.BlockSpec((dt, d), lambda i: (0, 0)),
      ],
      out_specs=pl.BlockSpec((block_rows, d), lambda i: (i, 0)),
      out_shape=jax.ShapeDtypeStruct((n, d), jnp.float32),
      compiler_params=pltpu.CompilerParams(
          dimension_semantics=("parallel",)),
  )(s, acc, tagg, W_tmp)


def kernel(x, edge_index, temporal_features, W_S, b_S, W_T, b_T, W_tmp, b_tmp):
  n, d = x.shape
  row = edge_index[0].astype(jnp.int32)
  col = edge_index[1].astype(jnp.int32)
  tf = temporal_features
  b_s2 = b_S.reshape(1, d).astype(jnp.float32)
  b_vec = (b_T + b_tmp).reshape(1, d).astype(jnp.float32)

  s_feat, y = _pre(x, W_S, W_T, b_s2, b_vec, block_rows=1000)

  n_pad = ((n + CHUNK - 1) // CHUNK) * CHUNK
  zacc = jnp.zeros((n_pad, d), jnp.float32)
  zt = jnp.zeros((n_pad, tf.shape[1]), jnp.float32)
  ids = jnp.arange(n_pad, dtype=jnp.int32)
  acc, tagg = _sc_agg(y, row, col, tf, zacc, zt, ids)

  return _post(s_feat, acc, tagg, W_tmp, block_rows=1000)


# speedup vs baseline: 2.3088x; 1.0487x over previous
import functools

import jax
import jax.numpy as jnp
from jax import lax
from jax.experimental import pallas as pl
from jax.experimental.pallas import tpu as pltpu
from jax.experimental.pallas import tpu_sc as plsc

NC = 2
NS = 16
NW = NC * NS
CHUNK = 128
RING = 2


def _pre_body(x_ref, ws_ref, wt_ref, bs_ref, bv_ref, s_ref, y_ref):
  xb = x_ref[...]
  s_ref[...] = jnp.dot(xb, ws_ref[...], preferred_element_type=jnp.float32) + bs_ref[...]
  y_ref[...] = jnp.dot(xb, wt_ref[...], preferred_element_type=jnp.float32) + bv_ref[...]


def _pre(x, W_S, W_T, b_S, b_vec, block_rows):
  n, d = x.shape
  grid = (n // block_rows,)
  out = jax.ShapeDtypeStruct((n, d), jnp.float32)
  return pl.pallas_call(
      _pre_body,
      grid=grid,
      in_specs=[
          pl.BlockSpec((block_rows, d), lambda i: (i, 0)),
          pl.BlockSpec((d, d), lambda i: (0, 0)),
          pl.BlockSpec((d, d), lambda i: (0, 0)),
          pl.BlockSpec((1, d), lambda i: (0, 0)),
          pl.BlockSpec((1, d), lambda i: (0, 0)),
      ],
      out_specs=[
          pl.BlockSpec((block_rows, d), lambda i: (i, 0)),
          pl.BlockSpec((block_rows, d), lambda i: (i, 0)),
      ],
      out_shape=[out, out],
      compiler_params=pltpu.CompilerParams(
          dimension_semantics=("parallel",)),
  )(x, W_S, W_T, b_S, b_vec)


def _sc_agg_body(n_pad, q0, extra, y_hbm, ei_hbm, tf_hbm,
                 ids_hbm, acc_out, tagg_out,
                 acc_sp, tagg_sp, ids_v,
                 col_v0, col_v1, row_v0, row_v1,
                 rows_v0, rows_v1, tf_v0, tf_v1,
                 lsem0, lsem1, gsem0, gsem1, ssem0, ssem1):
  c = lax.axis_index("c")
  s = lax.axis_index("s")
  wid = c * NS + s
  col_v = (col_v0, col_v1)
  row_v = (row_v0, row_v1)
  rows_v = (rows_v0, rows_v1)
  tf_v = (tf_v0, tf_v1)
  lsem = (lsem0, lsem1)
  gsem = (gsem0, gsem1)
  ssem = (ssem0, ssem1)

  z16 = jnp.zeros((16,), jnp.float32)

  @pl.loop(0, CHUNK)
  def _zbuf(i):
    for j in range(8):
      rows_v0[i, pl.ds(j * 16, 16)] = z16
    tf_v0[i, :] = z16

  nzc = n_pad // CHUNK
  my_nz = (nzc - s + NS - 1) // NS

  @pl.loop(0, my_nz)
  def _zero(k):
    off = (s + k * NS) * CHUNK
    pltpu.sync_copy(ids_hbm.at[pl.ds(off, CHUNK)], ids_v)
    pltpu.sync_copy(rows_v0, acc_sp.at[ids_v])
    pltpu.sync_copy(tf_v0, tagg_sp.at[ids_v])

  plsc.subcore_barrier()

  base = wid * q0

  def issue_loads(g, b):
    off = (base + g) * CHUNK
    pltpu.async_copy(ei_hbm.at[1, pl.ds(off, CHUNK)], col_v[b], lsem[b])
    pltpu.async_copy(ei_hbm.at[0, pl.ds(off, CHUNK)], row_v[b], lsem[b])
    pltpu.async_copy(tf_hbm.at[pl.ds(off, CHUNK), :], tf_v[b], lsem[b])

  def wait_loads(b):
    pltpu.make_async_copy(ei_hbm.at[1, pl.ds(0, CHUNK)], col_v[b], lsem[b]).wait()
    pltpu.make_async_copy(ei_hbm.at[0, pl.ds(0, CHUNK)], row_v[b], lsem[b]).wait()
    pltpu.make_async_copy(tf_hbm.at[pl.ds(0, CHUNK), :], tf_v[b], lsem[b]).wait()

  def drain_scatters(b):
    pltpu.make_async_copy(rows_v[b], acc_sp.at[row_v[b]], ssem[b]).wait()
    pltpu.make_async_copy(tf_v[b], tagg_sp.at[row_v[b]], ssem[b]).wait()

  issue_loads(0, 0)

  @pl.loop(0, q0 // RING)
  def _pipe(p):
    for r in range(RING):
      g = RING * p + r
      b = r
      ob = 1 - r
      wait_loads(b)
      pltpu.async_copy(y_hbm.at[col_v[b]], rows_v[b], gsem[b])
      if r == 0:
        @pl.when(p > 0)
        def _():
          drain_scatters(ob)
        issue_loads(g + 1, ob)
      else:
        drain_scatters(ob)

        @pl.when(g + 1 < q0)
        def _():
          issue_loads(g + 1, ob)
      pltpu.make_async_copy(y_hbm.at[col_v[b]], rows_v[b], gsem[b]).wait()
      pltpu.async_copy(rows_v[b], acc_sp.at[row_v[b]], ssem[b], add=True)
      pltpu.async_copy(tf_v[b], tagg_sp.at[row_v[b]], ssem[b], add=True)

  drain_scatters((q0 - 1) % RING)

  if extra:
    @pl.when(wid < extra)
    def _extra():
      off = (q0 * NW + wid) * CHUNK
      pltpu.sync_copy(ei_hbm.at[1, pl.ds(off, CHUNK)], col_v0)
      pltpu.sync_copy(ei_hbm.at[0, pl.ds(off, CHUNK)], row_v0)
      pltpu.sync_copy(tf_hbm.at[pl.ds(off, CHUNK), :], tf_v0)
      pltpu.sync_copy(y_hbm.at[col_v0], rows_v0)
      pltpu.sync_copy(rows_v0, acc_sp.at[row_v0], add=True)
      pltpu.sync_copy(tf_v0, tagg_sp.at[row_v0], add=True)

  plsc.subcore_barrier()

  @pl.loop(0, my_nz)
  def _wout(k):
    off = (s + k * NS) * CHUNK
    pltpu.sync_copy(ids_hbm.at[pl.ds(off, CHUNK)], ids_v)
    pltpu.sync_copy(acc_sp.at[ids_v], rows_v0)
    pltpu.sync_copy(tagg_sp.at[ids_v], tf_v0)
    pltpu.sync_copy(rows_v0, acc_out.at[c, pl.ds(off, CHUNK), :])
    pltpu.sync_copy(tf_v0, tagg_out.at[c, pl.ds(off, CHUNK), :])


def _sc_agg(y, ei, tf, ids, n_pad):
  d = y.shape[1]
  e = ei.shape[1]
  dt = tf.shape[1]
  assert e % CHUNK == 0 and n_pad % CHUNK == 0
  nec = e // CHUNK
  q0 = (nec // NW) // RING * RING
  extra = nec - q0 * NW
  assert 0 <= extra < 2 * NW and q0 > 0
  mesh = plsc.VectorSubcoreMesh(core_axis_name="c", subcore_axis_name="s")
  kern = pl.kernel(
      functools.partial(_sc_agg_body, n_pad, q0, extra),
      out_type=[
          jax.ShapeDtypeStruct((NC, n_pad, d), jnp.float32),
          jax.ShapeDtypeStruct((NC, n_pad, dt), jnp.float32),
      ],
      mesh=mesh,
      compiler_params=pltpu.CompilerParams(use_tc_tiling_on_sc=False),
      scratch_types=(
          [pltpu.VMEM_SHARED((n_pad, d), jnp.float32),
           pltpu.VMEM_SHARED((n_pad, dt), jnp.float32),
           pltpu.VMEM((CHUNK,), jnp.int32)]
          + [pltpu.VMEM((CHUNK,), jnp.int32)] * 4
          + [pltpu.VMEM((CHUNK, d), jnp.float32)] * 2
          + [pltpu.VMEM((CHUNK, dt), jnp.float32)] * 2
          + [pltpu.SemaphoreType.DMA] * 6
      ),
  )
  return kern(y, ei, tf, ids)


def _post_body(s_ref, acc_ref, tagg_ref, wt_ref, o_ref):
  agg = acc_ref[0] + acc_ref[1]
  tg = tagg_ref[0] + tagg_ref[1]
  msg = agg + jnp.dot(tg, wt_ref[...], preferred_element_type=jnp.float32)
  o_ref[...] = jnp.maximum(s_ref[...] + msg, 0.0)


def _post(s, acc, tagg, W_tmp, block_rows):
  n, d = s.shape
  dt = W_tmp.shape[0]
  grid = (n // block_rows,)
  return pl.pallas_call(
      _post_body,
      grid=grid,
      in_specs=[
          pl.BlockSpec((block_rows, d), lambda i: (i, 0)),
          pl.BlockSpec((NC, block_rows, d), lambda i: (0, i, 0)),
          pl.BlockSpec((NC, block_rows, dt), lambda i: (0, i, 0)),
          pl.BlockSpec((dt, d), lambda i: (0, 0)),
      ],
      out_specs=pl.BlockSpec((block_rows, d), lambda i: (i, 0)),
      out_shape=jax.ShapeDtypeStruct((n, d), jnp.float32),
      compiler_params=pltpu.CompilerParams(
          dimension_semantics=("parallel",)),
  )(s, acc, tagg, W_tmp)


def kernel(x, edge_index, temporal_features, W_S, b_S, W_T, b_T, W_tmp, b_tmp):
  n, d = x.shape
  ei = edge_index.astype(jnp.int32)
  tf = temporal_features
  b_s2 = b_S.reshape(1, d).astype(jnp.float32)
  b_vec = (b_T + b_tmp).reshape(1, d).astype(jnp.float32)

  s_feat, y = _pre(x, W_S, W_T, b_s2, b_vec, block_rows=1000)

  n_pad = ((n + CHUNK - 1) // CHUNK) * CHUNK
  ids = jnp.arange(n_pad, dtype=jnp.int32)
  acc, tagg = _sc_agg(y, ei, tf, ids, n_pad)

  return _post(s_feat, acc, tagg, W_tmp, block_rows=1000)
